# Initial kernel scaffold; baseline (speedup 1.0000x reference)
import jax
import jax.numpy as jnp
from jax.experimental import pallas as pl


def kernel(z, Wp1, bp1, Wp2, bp2, ln_g, ln_b, Wd, bd, codebooks):
    raise NotImplementedError("write your pallas kernel here")


# speedup vs baseline: 1.0960x; 1.0960x over previous
import math

import jax
import jax.numpy as jnp
import numpy as np
from jax.experimental import pallas as pl
from jax.experimental.pallas import tpu as pltpu

H = 4
K = 1024
D = 32
HID = 1024
B = 4096
TEMP = 1.0
HD = H * D
HK = H * K
TD = 2 * D
HTD = H * TD

BB = 256
NB = B // BB

PAIRS = [(i, j) for i in range(H) for j in range(i + 1, H)]
NPAIR = len(PAIRS)

_LOGK = float(np.log(K))

_MAVG_NP = np.zeros((HD, H), np.float32)
_MEXP_NP = np.zeros((H, HD), np.float32)
_MSUM_NP = np.zeros((HD, H), np.float32)
_EXPK_NP = np.zeros((H, HK), np.float32)
for _h in range(H):
    _MAVG_NP[_h * D:(_h + 1) * D, _h] = 1.0 / D
    _MEXP_NP[_h, _h * D:(_h + 1) * D] = 1.0
    _MSUM_NP[_h * D:(_h + 1) * D, _h] = 1.0
    _EXPK_NP[_h, _h * K:(_h + 1) * K] = 1.0


def _gelu_exact(x):
    return 0.5 * x * (1.0 + jax.lax.erf(x * (1.0 / math.sqrt(2.0))))


def _blockdiag(w):
    h, a, b = w.shape
    out = jnp.zeros((h * a, h * b), w.dtype)
    for i in range(h):
        out = jax.lax.dynamic_update_slice(out, w[i], (i * a, i * b))
    return out


def _main_body(z_ref, w1_ref, b1_ref, w2_ref, b2_ref, g_ref, lb_ref,
               wd_ref, bd_ref, cbt_ref, cbbd_ref, mavg_ref, mexp_ref,
               msum_ref, expk_ref,
               zq_ref, zh_ref, i0_ref, i1_ref, i2_ref, i3_ref,
               hist_ref, scal_ref, psum_ref, csum_ref):
    step = pl.program_id(0)

    @pl.when(step == 0)
    def _init():
        psum_ref[...] = jnp.zeros_like(psum_ref)
        csum_ref[...] = jnp.zeros_like(csum_ref)
        hist_ref[...] = jnp.zeros_like(hist_ref)
        scal_ref[...] = jnp.zeros_like(scal_ref)

    zb = z_ref[...]
    h1 = jnp.dot(zb, w1_ref[...], preferred_element_type=jnp.float32) + b1_ref[...]
    h1 = _gelu_exact(h1)
    h2 = jnp.dot(h1, w2_ref[...], preferred_element_type=jnp.float32) + b2_ref[...]
    mu = jnp.dot(jnp.dot(h2, mavg_ref[...], preferred_element_type=jnp.float32, precision=jax.lax.Precision.HIGHEST),
                 mexp_ref[...], preferred_element_type=jnp.float32, precision=jax.lax.Precision.HIGHEST)
    cent = h2 - mu
    var = jnp.dot(jnp.dot(cent * cent, mavg_ref[...], preferred_element_type=jnp.float32, precision=jax.lax.Precision.HIGHEST),
                  mexp_ref[...], preferred_element_type=jnp.float32, precision=jax.lax.Precision.HIGHEST)
    zh = cent / jnp.sqrt(var + 1e-5) * g_ref[...] + lb_ref[...]
    zh_ref[...] = zh

    cbt = cbt_ref[...]
    csq = jnp.dot(jnp.ones((1, HD), jnp.float32), cbt * cbt,
                  preferred_element_type=jnp.float32, precision=jax.lax.Precision.HIGHEST)
    zc = jnp.dot(zh, cbt, preferred_element_type=jnp.float32)
    zsq_h = jnp.dot(zh * zh, msum_ref[...], preferred_element_type=jnp.float32, precision=jax.lax.Precision.HIGHEST)
    zsq = jnp.dot(zsq_h, expk_ref[...], preferred_element_type=jnp.float32, precision=jax.lax.Precision.HIGHEST)
    d = (zsq + csq) - 2.0 * zc

    idx_refs = (i0_ref, i1_ref, i2_ref, i3_ref)
    oh_parts = []
    ps_parts = []
    for h in range(H):
        dh = d[:, h * K:(h + 1) * K]
        idx = jnp.argmin(dh, axis=1).astype(jnp.int32)
        idx_col = idx.reshape(BB, 1)
        idx_refs[h][...] = idx_col
        iota = jax.lax.broadcasted_iota(jnp.int32, (BB, K), 1)
        oh_parts.append((iota == idx_col).astype(jnp.float32))
        m = jnp.min(dh, axis=1, keepdims=True)
        e = jnp.exp((m - dh) * TEMP)
        p = e / jnp.sum(e, axis=1, keepdims=True)
        ps_parts.append(jnp.sum(p, axis=0, keepdims=True))
    oh_cat = jnp.concatenate(oh_parts, axis=1)
    psum_ref[...] += jnp.concatenate(ps_parts, axis=1)
    hist_ref[...] += jnp.sum(oh_cat, axis=0, keepdims=True)

    zq = jnp.dot(oh_cat, cbbd_ref[...], preferred_element_type=jnp.float32, precision=jax.lax.Precision.HIGHEST)
    dec = jnp.dot(zq, wd_ref[...], preferred_element_type=jnp.float32) + bd_ref[...]
    proj = jnp.dot(zh, wd_ref[...], preferred_element_type=jnp.float32) + bd_ref[...]
    zq_ref[...] = dec
    diff = proj - dec
    csum_ref[...] += jnp.sum(diff * diff)

    @pl.when(step == NB - 1)
    def _fin():
        avg_p = psum_ref[...] * (1.0 / B)
        ent = -jnp.sum(avg_p * jnp.log(avg_p + 1e-8)) * (1.0 / H)
        commit = csum_ref[0, 0] * (1.0 / (B * HID))
        ent_loss = 1.0 - ent * (1.0 / _LOGK)
        scal_ref[...] = jnp.concatenate(
            [commit.reshape(1, 1), ent_loss.reshape(1, 1),
             jnp.zeros((1, 6), jnp.float32)], axis=1)


def _ortho_body(idxi_ref, idxj_ref, pi_ref, pj_ref, out_ref, co_ref, acc_ref):
    p = pl.program_id(0)
    b = pl.program_id(1)

    @pl.when(jnp.logical_and(p == 0, b == 0))
    def _init():
        acc_ref[...] = jnp.zeros_like(acc_ref)

    ii = idxi_ref[0]
    jj = idxj_ref[0]
    iota_col = jax.lax.broadcasted_iota(jnp.int32, (K, BB), 0)
    ohi_t = (iota_col == ii).astype(jnp.bfloat16)
    iota_row = jax.lax.broadcasted_iota(jnp.int32, (BB, K), 1)
    ohj = (iota_row == jj).astype(jnp.bfloat16)
    contrib = jnp.dot(ohi_t, ohj, preferred_element_type=jnp.float32)

    @pl.when(b == 0)
    def _reset():
        co_ref[...] = jnp.zeros_like(co_ref)

    co_ref[...] += contrib

    @pl.when(b == NB - 1)
    def _pair_done():
        pi = pi_ref[0]
        pj = pj_ref[0]
        t = jnp.abs(co_ref[...] * (1.0 / (B + 1e-8)) - pi * pj)
        acc_ref[...] += jnp.sum(t)

    @pl.when(jnp.logical_and(p == NPAIR - 1, b == NB - 1))
    def _fin():
        out_ref[...] = acc_ref[...] * (1.0 / (K * K * NPAIR))


def kernel(z, Wp1, bp1, Wp2, bp2, ln_g, ln_b, Wd, bd, codebooks):
    w1 = jnp.transpose(Wp1, (1, 0, 2)).reshape(HID, HTD)
    b1 = bp1.reshape(1, HTD)
    w2 = _blockdiag(Wp2)
    b2 = bp2.reshape(1, HD)
    g = ln_g.reshape(1, HD)
    lb = ln_b.reshape(1, HD)
    wd = _blockdiag(Wd)
    bdc = bd.reshape(1, HID)
    cbt = _blockdiag(jnp.transpose(codebooks, (0, 2, 1)))
    cbbd = _blockdiag(codebooks)
    mavg = jnp.asarray(_MAVG_NP)
    mexp = jnp.asarray(_MEXP_NP)
    msum = jnp.asarray(_MSUM_NP)
    expk = jnp.asarray(_EXPK_NP)

    const = lambda i: (0, 0)
    row = lambda i: (i, 0)
    outs = pl.pallas_call(
        _main_body,
        grid=(NB,),
        in_specs=[
            pl.BlockSpec((BB, HID), row),
            pl.BlockSpec((HID, HTD), const),
            pl.BlockSpec((1, HTD), const),
            pl.BlockSpec((HTD, HD), const),
            pl.BlockSpec((1, HD), const),
            pl.BlockSpec((1, HD), const),
            pl.BlockSpec((1, HD), const),
            pl.BlockSpec((HD, HID), const),
            pl.BlockSpec((1, HID), const),
            pl.BlockSpec((HD, HK), const),
            pl.BlockSpec((HK, HD), const),
            pl.BlockSpec((HD, H), const),
            pl.BlockSpec((H, HD), const),
            pl.BlockSpec((HD, H), const),
            pl.BlockSpec((H, HK), const),
        ],
        out_specs=[
            pl.BlockSpec((BB, HID), row),
            pl.BlockSpec((BB, HD), row),
            pl.BlockSpec((BB, 1), row),
            pl.BlockSpec((BB, 1), row),
            pl.BlockSpec((BB, 1), row),
            pl.BlockSpec((BB, 1), row),
            pl.BlockSpec((1, HK), const),
            pl.BlockSpec((1, 8), const),
        ],
        out_shape=[
            jax.ShapeDtypeStruct((B, HID), jnp.float32),
            jax.ShapeDtypeStruct((B, HD), jnp.float32),
            jax.ShapeDtypeStruct((B, 1), jnp.int32),
            jax.ShapeDtypeStruct((B, 1), jnp.int32),
            jax.ShapeDtypeStruct((B, 1), jnp.int32),
            jax.ShapeDtypeStruct((B, 1), jnp.int32),
            jax.ShapeDtypeStruct((1, HK), jnp.float32),
            jax.ShapeDtypeStruct((1, 8), jnp.float32),
        ],
        scratch_shapes=[
            pltpu.VMEM((1, HK), jnp.float32),
            pltpu.VMEM((1, 1), jnp.float32),
        ],
    )(z, w1, b1, w2, b2, g, lb, wd, bdc, cbt, cbbd, mavg, mexp, msum, expk)
    zq_st, zh_flat, i0, i1, i2, i3, hist, scal = outs

    indices = jnp.concatenate([i0, i1, i2, i3], axis=1)

    idx_i = jnp.stack([indices[:, i] for (i, j) in PAIRS])
    idx_j = jnp.stack([indices[:, j] for (i, j) in PAIRS])
    idx_i3 = idx_i.reshape(NPAIR * NB, 1, BB)
    idx_j3 = idx_j.reshape(NPAIR * NB, BB, 1)
    p_head = hist.reshape(H, K) * (1.0 / B)
    p_i = jnp.stack([p_head[i] for (i, j) in PAIRS]).reshape(NPAIR, K, 1)
    p_j = jnp.stack([p_head[j] for (i, j) in PAIRS]).reshape(NPAIR, 1, K)

    ortho = pl.pallas_call(
        _ortho_body,
        grid=(NPAIR, NB),
        in_specs=[
            pl.BlockSpec((1, 1, BB), lambda p, b: (p * NB + b, 0, 0)),
            pl.BlockSpec((1, BB, 1), lambda p, b: (p * NB + b, 0, 0)),
            pl.BlockSpec((1, K, 1), lambda p, b: (p, 0, 0)),
            pl.BlockSpec((1, 1, K), lambda p, b: (p, 0, 0)),
        ],
        out_specs=pl.BlockSpec((1, 1), lambda p, b: (0, 0)),
        out_shape=jax.ShapeDtypeStruct((1, 1), jnp.float32),
        scratch_shapes=[
            pltpu.VMEM((K, K), jnp.float32),
            pltpu.VMEM((1, 1), jnp.float32),
        ],
    )(idx_i3, idx_j3, p_i, p_j)

    commit = scal[0, 0]
    entropy_loss = scal[0, 1]
    ortho_loss = ortho[0, 0]
    z_heads = zh_flat.reshape(B, H, D)
    return (zq_st, indices, commit, commit, entropy_loss, ortho_loss, z_heads)


# speedup vs baseline: 1.4791x; 1.3495x over previous
import math

import jax
import jax.numpy as jnp
import numpy as np
from jax.experimental import pallas as pl
from jax.experimental.pallas import tpu as pltpu

H = 4
K = 1024
D = 32
HID = 1024
B = 4096
TEMP = 1.0
HD = H * D
HK = H * K
TD = 2 * D
HTD = H * TD

BB = 256
NB = B // BB

PAIRS = [(i, j) for i in range(H) for j in range(i + 1, H)]
NPAIR = len(PAIRS)

_LOGK = float(np.log(K))

_MAVG_NP = np.zeros((HD, H), np.float32)
_MEXP_NP = np.zeros((H, HD), np.float32)
_MSUM_NP = np.zeros((HD, H), np.float32)
_EXPK_NP = np.zeros((H, HK), np.float32)
for _h in range(H):
    _MAVG_NP[_h * D:(_h + 1) * D, _h] = 1.0 / D
    _MEXP_NP[_h, _h * D:(_h + 1) * D] = 1.0
    _MSUM_NP[_h * D:(_h + 1) * D, _h] = 1.0
    _EXPK_NP[_h, _h * K:(_h + 1) * K] = 1.0


def _gelu_exact(x):
    return 0.5 * x * (1.0 + jax.lax.erf(x * (1.0 / math.sqrt(2.0))))


def _blockdiag(w):
    h, a, b = w.shape
    out = jnp.zeros((h * a, h * b), w.dtype)
    for i in range(h):
        out = jax.lax.dynamic_update_slice(out, w[i], (i * a, i * b))
    return out


def _main_body(z_ref, w1_ref, b1_ref, w2_ref, b2_ref, g_ref, lb_ref,
               wd_ref, bd_ref, cbt_ref, cbhi_ref, cblo_ref, mavg_ref,
               mexp_ref, msum_ref,
               zq_ref, zh_ref, i0_ref, i1_ref, i2_ref, i3_ref,
               hist_ref, scal_ref, psum_ref, csum_ref, csq_ref):
    step = pl.program_id(0)

    @pl.when(step == 0)
    def _init():
        psum_ref[...] = jnp.zeros_like(psum_ref)
        csum_ref[...] = jnp.zeros_like(csum_ref)
        hist_ref[...] = jnp.zeros_like(hist_ref)
        scal_ref[...] = jnp.zeros_like(scal_ref)
        cbt0 = cbt_ref[...]
        csq_ref[...] = jnp.dot(jnp.ones((1, HD), jnp.float32), cbt0 * cbt0,
                               preferred_element_type=jnp.float32,
                               precision=jax.lax.Precision.HIGHEST)

    zb = z_ref[...]
    h1 = jnp.dot(zb, w1_ref[...], preferred_element_type=jnp.float32) + b1_ref[...]
    h1 = _gelu_exact(h1)
    h2 = jnp.dot(h1, w2_ref[...], preferred_element_type=jnp.float32) + b2_ref[...]
    mu = jnp.dot(jnp.dot(h2, mavg_ref[...], preferred_element_type=jnp.float32, precision=jax.lax.Precision.HIGHEST),
                 mexp_ref[...], preferred_element_type=jnp.float32, precision=jax.lax.Precision.HIGHEST)
    cent = h2 - mu
    var = jnp.dot(jnp.dot(cent * cent, mavg_ref[...], preferred_element_type=jnp.float32, precision=jax.lax.Precision.HIGHEST),
                  mexp_ref[...], preferred_element_type=jnp.float32, precision=jax.lax.Precision.HIGHEST)
    zh = cent / jnp.sqrt(var + 1e-5) * g_ref[...] + lb_ref[...]
    zh_ref[...] = zh

    csq = csq_ref[...]
    zc = jnp.dot(zh, cbt_ref[...], preferred_element_type=jnp.float32)
    zsq_h = jnp.dot(zh * zh, msum_ref[...], preferred_element_type=jnp.float32, precision=jax.lax.Precision.HIGHEST)

    idx_refs = (i0_ref, i1_ref, i2_ref, i3_ref)
    oh_parts = []
    ps_parts = []
    for h in range(H):
        dh = (zsq_h[:, h:h + 1] + csq[:, h * K:(h + 1) * K]) - 2.0 * zc[:, h * K:(h + 1) * K]
        idx = jnp.argmin(dh, axis=1).astype(jnp.int32)
        idx_col = idx.reshape(BB, 1)
        idx_refs[h][...] = idx_col
        iota = jax.lax.broadcasted_iota(jnp.int32, (BB, K), 1)
        oh_parts.append((iota == idx_col).astype(jnp.float32))
        m = jnp.min(dh, axis=1, keepdims=True)
        e = jnp.exp((m - dh) * TEMP)
        p = e / jnp.sum(e, axis=1, keepdims=True)
        ps_parts.append(jnp.sum(p, axis=0, keepdims=True))
    oh_cat = jnp.concatenate(oh_parts, axis=1)
    psum_ref[...] += jnp.concatenate(ps_parts, axis=1)
    hist_ref[...] += jnp.sum(oh_cat, axis=0, keepdims=True)

    oh_bf = oh_cat.astype(jnp.bfloat16)
    zq = (jnp.dot(oh_bf, cbhi_ref[...], preferred_element_type=jnp.float32)
          + jnp.dot(oh_bf, cblo_ref[...], preferred_element_type=jnp.float32))
    dec = jnp.dot(zq, wd_ref[...], preferred_element_type=jnp.float32) + bd_ref[...]
    proj = jnp.dot(zh, wd_ref[...], preferred_element_type=jnp.float32) + bd_ref[...]
    zq_ref[...] = dec
    diff = proj - dec
    csum_ref[...] += jnp.sum(diff * diff)

    @pl.when(step == NB - 1)
    def _fin():
        avg_p = psum_ref[...] * (1.0 / B)
        ent = -jnp.sum(avg_p * jnp.log(avg_p + 1e-8)) * (1.0 / H)
        commit = csum_ref[0, 0] * (1.0 / (B * HID))
        ent_loss = 1.0 - ent * (1.0 / _LOGK)
        scal_ref[...] = jnp.concatenate(
            [commit.reshape(1, 1), ent_loss.reshape(1, 1),
             jnp.zeros((1, 6), jnp.float32)], axis=1)


def _ortho_body(idxi_ref, idxj_ref, pi_ref, pj_ref, out_ref, co_ref, acc_ref):
    p = pl.program_id(0)
    b = pl.program_id(1)

    @pl.when(jnp.logical_and(p == 0, b == 0))
    def _init():
        acc_ref[...] = jnp.zeros_like(acc_ref)

    ii = idxi_ref[0]
    jj = idxj_ref[0]
    iota_col = jax.lax.broadcasted_iota(jnp.int32, (K, BB), 0)
    ohi_t = (iota_col == ii).astype(jnp.bfloat16)
    iota_row = jax.lax.broadcasted_iota(jnp.int32, (BB, K), 1)
    ohj = (iota_row == jj).astype(jnp.bfloat16)
    contrib = jnp.dot(ohi_t, ohj, preferred_element_type=jnp.float32)

    @pl.when(b == 0)
    def _reset():
        co_ref[...] = jnp.zeros_like(co_ref)

    co_ref[...] += contrib

    @pl.when(b == NB - 1)
    def _pair_done():
        pi = pi_ref[0]
        pj = pj_ref[0]
        t = jnp.abs(co_ref[...] * (1.0 / (B + 1e-8)) - pi * pj)
        acc_ref[...] += jnp.sum(t)

    @pl.when(jnp.logical_and(p == NPAIR - 1, b == NB - 1))
    def _fin():
        out_ref[...] = acc_ref[...] * (1.0 / (K * K * NPAIR))


def kernel(z, Wp1, bp1, Wp2, bp2, ln_g, ln_b, Wd, bd, codebooks):
    w1 = jnp.transpose(Wp1, (1, 0, 2)).reshape(HID, HTD)
    b1 = bp1.reshape(1, HTD)
    w2 = _blockdiag(Wp2)
    b2 = bp2.reshape(1, HD)
    g = ln_g.reshape(1, HD)
    lb = ln_b.reshape(1, HD)
    wd = _blockdiag(Wd)
    bdc = bd.reshape(1, HID)
    cbt = _blockdiag(jnp.transpose(codebooks, (0, 2, 1)))
    cbbd = _blockdiag(codebooks)
    cbhi = cbbd.astype(jnp.bfloat16)
    cblo = (cbbd - cbhi.astype(jnp.float32)).astype(jnp.bfloat16)
    mavg = jnp.asarray(_MAVG_NP)
    mexp = jnp.asarray(_MEXP_NP)
    msum = jnp.asarray(_MSUM_NP)

    const = lambda i: (0, 0)
    row = lambda i: (i, 0)
    outs = pl.pallas_call(
        _main_body,
        grid=(NB,),
        in_specs=[
            pl.BlockSpec((BB, HID), row),
            pl.BlockSpec((HID, HTD), const),
            pl.BlockSpec((1, HTD), const),
            pl.BlockSpec((HTD, HD), const),
            pl.BlockSpec((1, HD), const),
            pl.BlockSpec((1, HD), const),
            pl.BlockSpec((1, HD), const),
            pl.BlockSpec((HD, HID), const),
            pl.BlockSpec((1, HID), const),
            pl.BlockSpec((HD, HK), const),
            pl.BlockSpec((HK, HD), const),
            pl.BlockSpec((HK, HD), const),
            pl.BlockSpec((HD, H), const),
            pl.BlockSpec((H, HD), const),
            pl.BlockSpec((HD, H), const),
        ],
        out_specs=[
            pl.BlockSpec((BB, HID), row),
            pl.BlockSpec((BB, HD), row),
            pl.BlockSpec((BB, 1), row),
            pl.BlockSpec((BB, 1), row),
            pl.BlockSpec((BB, 1), row),
            pl.BlockSpec((BB, 1), row),
            pl.BlockSpec((1, HK), const),
            pl.BlockSpec((1, 8), const),
        ],
        out_shape=[
            jax.ShapeDtypeStruct((B, HID), jnp.float32),
            jax.ShapeDtypeStruct((B, HD), jnp.float32),
            jax.ShapeDtypeStruct((B, 1), jnp.int32),
            jax.ShapeDtypeStruct((B, 1), jnp.int32),
            jax.ShapeDtypeStruct((B, 1), jnp.int32),
            jax.ShapeDtypeStruct((B, 1), jnp.int32),
            jax.ShapeDtypeStruct((1, HK), jnp.float32),
            jax.ShapeDtypeStruct((1, 8), jnp.float32),
        ],
        scratch_shapes=[
            pltpu.VMEM((1, HK), jnp.float32),
            pltpu.VMEM((1, 1), jnp.float32),
            pltpu.VMEM((1, HK), jnp.float32),
        ],
    )(z, w1, b1, w2, b2, g, lb, wd, bdc, cbt, cbhi, cblo, mavg, mexp, msum)
    zq_st, zh_flat, i0, i1, i2, i3, hist, scal = outs

    indices = jnp.concatenate([i0, i1, i2, i3], axis=1)

    idx_i = jnp.stack([indices[:, i] for (i, j) in PAIRS])
    idx_j = jnp.stack([indices[:, j] for (i, j) in PAIRS])
    idx_i3 = idx_i.reshape(NPAIR * NB, 1, BB)
    idx_j3 = idx_j.reshape(NPAIR * NB, BB, 1)
    p_head = hist.reshape(H, K) * (1.0 / B)
    p_i = jnp.stack([p_head[i] for (i, j) in PAIRS]).reshape(NPAIR, K, 1)
    p_j = jnp.stack([p_head[j] for (i, j) in PAIRS]).reshape(NPAIR, 1, K)

    ortho = pl.pallas_call(
        _ortho_body,
        grid=(NPAIR, NB),
        in_specs=[
            pl.BlockSpec((1, 1, BB), lambda p, b: (p * NB + b, 0, 0)),
            pl.BlockSpec((1, BB, 1), lambda p, b: (p * NB + b, 0, 0)),
            pl.BlockSpec((1, K, 1), lambda p, b: (p, 0, 0)),
            pl.BlockSpec((1, 1, K), lambda p, b: (p, 0, 0)),
        ],
        out_specs=pl.BlockSpec((1, 1), lambda p, b: (0, 0)),
        out_shape=jax.ShapeDtypeStruct((1, 1), jnp.float32),
        scratch_shapes=[
            pltpu.VMEM((K, K), jnp.float32),
            pltpu.VMEM((1, 1), jnp.float32),
        ],
    )(idx_i3, idx_j3, p_i, p_j)

    commit = scal[0, 0]
    entropy_loss = scal[0, 1]
    ortho_loss = ortho[0, 0]
    z_heads = zh_flat.reshape(B, H, D)
    return (zq_st, indices, commit, commit, entropy_loss, ortho_loss, z_heads)


# speedup vs baseline: 1.4837x; 1.0031x over previous
import math

import jax
import jax.numpy as jnp
import numpy as np
from jax.experimental import pallas as pl
from jax.experimental.pallas import tpu as pltpu

H = 4
K = 1024
D = 32
HID = 1024
B = 4096
TEMP = 1.0
HD = H * D
HK = H * K
TD = 2 * D
HTD = H * TD

BB = 256
NB = B // BB

PAIRS = [(i, j) for i in range(H) for j in range(i + 1, H)]
NPAIR = len(PAIRS)

_LOGK = float(np.log(K))

_MAVG_NP = np.zeros((HD, H), np.float32)
_MEXP_NP = np.zeros((H, HD), np.float32)
_MSUM_NP = np.zeros((HD, H), np.float32)
_EXPK_NP = np.zeros((H, HK), np.float32)
for _h in range(H):
    _MAVG_NP[_h * D:(_h + 1) * D, _h] = 1.0 / D
    _MEXP_NP[_h, _h * D:(_h + 1) * D] = 1.0
    _MSUM_NP[_h * D:(_h + 1) * D, _h] = 1.0
    _EXPK_NP[_h, _h * K:(_h + 1) * K] = 1.0


def _gelu_exact(x):
    return 0.5 * x * (1.0 + jax.lax.erf(x * (1.0 / math.sqrt(2.0))))


def _blockdiag(w):
    h, a, b = w.shape
    out = jnp.zeros((h * a, h * b), w.dtype)
    for i in range(h):
        out = jax.lax.dynamic_update_slice(out, w[i], (i * a, i * b))
    return out


def _main_body(z_ref, w1_ref, b1_ref, w2_ref, b2_ref, g_ref, lb_ref,
               wd_ref, bd_ref, cbt_ref, cbhi_ref, cblo_ref, mavg_ref,
               mexp_ref, msum_ref,
               zq_ref, zh_ref, i0_ref, i1_ref, i2_ref, i3_ref,
               hist_ref, scal_ref, psum_ref, csum_ref, csq_ref):
    step = pl.program_id(0)

    @pl.when(step == 0)
    def _init():
        psum_ref[...] = jnp.zeros_like(psum_ref)
        csum_ref[...] = jnp.zeros_like(csum_ref)
        hist_ref[...] = jnp.zeros_like(hist_ref)
        scal_ref[...] = jnp.zeros_like(scal_ref)
        cbt0 = cbt_ref[...]
        csq_ref[...] = jnp.dot(jnp.ones((1, HD), jnp.float32), cbt0 * cbt0,
                               preferred_element_type=jnp.float32,
                               precision=jax.lax.Precision.HIGHEST)

    zb = z_ref[...]
    h1 = jnp.dot(zb, w1_ref[...], preferred_element_type=jnp.float32) + b1_ref[...]
    h1 = _gelu_exact(h1)
    h2 = jnp.dot(h1, w2_ref[...], preferred_element_type=jnp.float32) + b2_ref[...]
    mu = jnp.dot(jnp.dot(h2, mavg_ref[...], preferred_element_type=jnp.float32, precision=jax.lax.Precision.HIGHEST),
                 mexp_ref[...], preferred_element_type=jnp.float32, precision=jax.lax.Precision.HIGHEST)
    cent = h2 - mu
    var = jnp.dot(jnp.dot(cent * cent, mavg_ref[...], preferred_element_type=jnp.float32, precision=jax.lax.Precision.HIGHEST),
                  mexp_ref[...], preferred_element_type=jnp.float32, precision=jax.lax.Precision.HIGHEST)
    zh = cent / jnp.sqrt(var + 1e-5) * g_ref[...] + lb_ref[...]
    zh_ref[...] = zh

    csq = csq_ref[...]
    zc = jnp.dot(zh, cbt_ref[...], preferred_element_type=jnp.float32)
    zsq_h = jnp.dot(zh * zh, msum_ref[...], preferred_element_type=jnp.float32, precision=jax.lax.Precision.HIGHEST)

    idx_refs = (i0_ref, i1_ref, i2_ref, i3_ref)
    oh_parts = []
    ps_parts = []
    for h in range(H):
        dh = (zsq_h[:, h:h + 1] + csq[:, h * K:(h + 1) * K]) - 2.0 * zc[:, h * K:(h + 1) * K]
        idx = jnp.argmin(dh, axis=1).astype(jnp.int32)
        idx_col = idx.reshape(BB, 1)
        idx_refs[h][...] = idx_col
        iota = jax.lax.broadcasted_iota(jnp.int32, (BB, K), 1)
        oh_parts.append((iota == idx_col).astype(jnp.float32))
        m = jnp.min(dh, axis=1, keepdims=True)
        e = jnp.exp((m - dh) * TEMP)
        p = e / jnp.sum(e, axis=1, keepdims=True)
        ps_parts.append(jnp.sum(p, axis=0, keepdims=True))
    oh_cat = jnp.concatenate(oh_parts, axis=1)
    psum_ref[...] += jnp.concatenate(ps_parts, axis=1)
    hist_ref[...] += jnp.sum(oh_cat, axis=0, keepdims=True)


    oh_bf = oh_cat.astype(jnp.bfloat16)
    zq = (jnp.dot(oh_bf, cbhi_ref[...], preferred_element_type=jnp.float32)
          + jnp.dot(oh_bf, cblo_ref[...], preferred_element_type=jnp.float32))
    dec = jnp.dot(zq, wd_ref[...], preferred_element_type=jnp.float32) + bd_ref[...]
    proj = jnp.dot(zh, wd_ref[...], preferred_element_type=jnp.float32) + bd_ref[...]
    zq_ref[...] = dec
    diff = proj - dec
    csum_ref[...] += jnp.sum(diff * diff)

    @pl.when(step == NB - 1)
    def _fin():
        avg_p = psum_ref[...] * (1.0 / B)
        ent = -jnp.sum(avg_p * jnp.log(avg_p + 1e-8)) * (1.0 / H)
        commit = csum_ref[0, 0] * (1.0 / (B * HID))
        ent_loss = 1.0 - ent * (1.0 / _LOGK)
        scal_ref[...] = jnp.concatenate(
            [commit.reshape(1, 1), ent_loss.reshape(1, 1),
             jnp.zeros((1, 6), jnp.float32)], axis=1)


def _ortho_body(idxi_ref, idxj_ref, pi_ref, pj_ref, out_ref, co_ref, acc_ref):
    p = pl.program_id(0)
    b = pl.program_id(1)

    @pl.when(jnp.logical_and(p == 0, b == 0))
    def _init():
        acc_ref[...] = jnp.zeros_like(acc_ref)

    ii = idxi_ref[0]
    jj = idxj_ref[0]
    iota_col = jax.lax.broadcasted_iota(jnp.int32, (K, BB), 0)
    ohi_t = (iota_col == ii).astype(jnp.bfloat16)
    iota_row = jax.lax.broadcasted_iota(jnp.int32, (BB, K), 1)
    ohj = (iota_row == jj).astype(jnp.bfloat16)
    contrib = jnp.dot(ohi_t, ohj, preferred_element_type=jnp.float32)

    @pl.when(b == 0)
    def _reset():
        co_ref[...] = jnp.zeros_like(co_ref)

    co_ref[...] += contrib

    @pl.when(b == NB - 1)
    def _pair_done():
        pi = pi_ref[0]
        pj = pj_ref[0]
        t = jnp.abs(co_ref[...] * (1.0 / (B + 1e-8)) - pi * pj)
        acc_ref[...] += jnp.sum(t)

    @pl.when(jnp.logical_and(p == NPAIR - 1, b == NB - 1))
    def _fin():
        out_ref[...] = acc_ref[...] * (1.0 / (K * K * NPAIR))


def kernel(z, Wp1, bp1, Wp2, bp2, ln_g, ln_b, Wd, bd, codebooks):
    w1 = jnp.transpose(Wp1, (1, 0, 2)).reshape(HID, HTD)
    b1 = bp1.reshape(1, HTD)
    w2 = _blockdiag(Wp2)
    b2 = bp2.reshape(1, HD)
    g = ln_g.reshape(1, HD)
    lb = ln_b.reshape(1, HD)
    wd = _blockdiag(Wd)
    bdc = bd.reshape(1, HID)
    cbt = _blockdiag(jnp.transpose(codebooks, (0, 2, 1)))
    cbbd = _blockdiag(codebooks)
    cbhi = cbbd.astype(jnp.bfloat16)
    cblo = (cbbd - cbhi.astype(jnp.float32)).astype(jnp.bfloat16)
    mavg = jnp.asarray(_MAVG_NP)
    mexp = jnp.asarray(_MEXP_NP)
    msum = jnp.asarray(_MSUM_NP)

    const = lambda i: (0, 0)
    row = lambda i: (i, 0)
    outs = pl.pallas_call(
        _main_body,
        grid=(NB,),
        in_specs=[
            pl.BlockSpec((BB, HID), row),
            pl.BlockSpec((HID, HTD), const),
            pl.BlockSpec((1, HTD), const),
            pl.BlockSpec((HTD, HD), const),
            pl.BlockSpec((1, HD), const),
            pl.BlockSpec((1, HD), const),
            pl.BlockSpec((1, HD), const),
            pl.BlockSpec((HD, HID), const),
            pl.BlockSpec((1, HID), const),
            pl.BlockSpec((HD, HK), const),
            pl.BlockSpec((HK, HD), const),
            pl.BlockSpec((HK, HD), const),
            pl.BlockSpec((HD, H), const),
            pl.BlockSpec((H, HD), const),
            pl.BlockSpec((HD, H), const),
        ],
        out_specs=[
            pl.BlockSpec((BB, HID), row),
            pl.BlockSpec((BB, HD), row),
            pl.BlockSpec((BB, 1), row),
            pl.BlockSpec((BB, 1), row),
            pl.BlockSpec((BB, 1), row),
            pl.BlockSpec((BB, 1), row),
            pl.BlockSpec((1, HK), const),
            pl.BlockSpec((1, 8), const),
        ],
        out_shape=[
            jax.ShapeDtypeStruct((B, HID), jnp.float32),
            jax.ShapeDtypeStruct((B, HD), jnp.float32),
            jax.ShapeDtypeStruct((B, 1), jnp.int32),
            jax.ShapeDtypeStruct((B, 1), jnp.int32),
            jax.ShapeDtypeStruct((B, 1), jnp.int32),
            jax.ShapeDtypeStruct((B, 1), jnp.int32),
            jax.ShapeDtypeStruct((1, HK), jnp.float32),
            jax.ShapeDtypeStruct((1, 8), jnp.float32),
        ],
        scratch_shapes=[
            pltpu.VMEM((1, HK), jnp.float32),
            pltpu.VMEM((1, 1), jnp.float32),
            pltpu.VMEM((1, HK), jnp.float32),
        ],
    )(z, w1, b1, w2, b2, g, lb, wd, bdc, cbt, cbhi, cblo, mavg, mexp, msum)
    zq_st, zh_flat, i0, i1, i2, i3, hist, scal = outs

    indices = jnp.concatenate([i0, i1, i2, i3], axis=1)

    idx_i = jnp.stack([indices[:, i] for (i, j) in PAIRS])
    idx_j = jnp.stack([indices[:, j] for (i, j) in PAIRS])
    idx_i3 = idx_i.reshape(NPAIR * NB, 1, BB)
    idx_j3 = idx_j.reshape(NPAIR * NB, BB, 1)
    p_head = hist.reshape(H, K) * (1.0 / B)
    p_i = jnp.stack([p_head[i] for (i, j) in PAIRS]).reshape(NPAIR, K, 1)
    p_j = jnp.stack([p_head[j] for (i, j) in PAIRS]).reshape(NPAIR, 1, K)

    ortho = pl.pallas_call(
        _ortho_body,
        grid=(NPAIR, NB),
        in_specs=[
            pl.BlockSpec((1, 1, BB), lambda p, b: (p * NB + b, 0, 0)),
            pl.BlockSpec((1, BB, 1), lambda p, b: (p * NB + b, 0, 0)),
            pl.BlockSpec((1, K, 1), lambda p, b: (p, 0, 0)),
            pl.BlockSpec((1, 1, K), lambda p, b: (p, 0, 0)),
        ],
        out_specs=pl.BlockSpec((1, 1), lambda p, b: (0, 0)),
        out_shape=jax.ShapeDtypeStruct((1, 1), jnp.float32),
        scratch_shapes=[
            pltpu.VMEM((K, K), jnp.float32),
            pltpu.VMEM((1, 1), jnp.float32),
        ],
    )(idx_i3, idx_j3, p_i, p_j)
    ortho_loss = ortho[0, 0]

    commit = scal[0, 0]
    entropy_loss = scal[0, 1]
    z_heads = zh_flat.reshape(B, H, D)
    return (zq_st, indices, commit, commit, entropy_loss, ortho_loss, z_heads)
